# baseline (speedup 1.0000x reference)
import jax
import jax.numpy as jnp
from jax.experimental import pallas as pl
from jax.experimental.pallas import tpu as pltpu

_N = 5000
_K = 12544
_H = 1024
_BN = 1280
_BK = 1792
_NK = _K // _BK
_NO = 2
_NI = 2
_FC = 2

_DN = (((1,), (1,)), ((), ()))


def _body(x_ref, w1_ref, b1_ref, w2_ref, b2_ref, wc_ref, bc_ref, wr_ref,
          br_ref, cls_ref, box_ref, acc_ref):
    k = pl.program_id(1)
    ni = pl.program_id(2)

    part = jax.lax.dot_general(
        x_ref[...], w1_ref[...], _DN, preferred_element_type=jnp.float32)

    @pl.when(k == 0)
    def _init():
        acc_ref[ni] = part

    @pl.when(k > 0)
    def _accum():
        acc_ref[ni] += part

    @pl.when(k == _NK - 1)
    def _finish():
        bc = _BN // _FC
        for c in range(_FC):
            rows = pl.ds(c * bc, bc)
            h1 = jnp.maximum(acc_ref[ni, rows, :] + b1_ref[...], 0.0)
            h2 = jax.lax.dot_general(
                h1, w2_ref[...], _DN, preferred_element_type=jnp.float32)
            h2 = jnp.maximum(h2 + b2_ref[...], 0.0)
            cls_ref[rows, :] = jax.lax.dot_general(
                h2, wc_ref[...], _DN,
                preferred_element_type=jnp.float32) + bc_ref[...]
            box_ref[rows, :] = jax.lax.dot_general(
                h2, wr_ref[...], _DN,
                preferred_element_type=jnp.float32) + br_ref[...]


def kernel(feature_vectors, W1, b1, W2, b2, Wc, bc, Wr, br):
    c1 = Wc.shape[0]
    c4 = Wr.shape[0]
    cls_out, box_out = pl.pallas_call(
        _body,
        grid=(_NO, _NK, _NI),
        in_specs=[
            pl.BlockSpec((_BN, _BK), lambda o, k, i: (o * _NI + i, k)),
            pl.BlockSpec((_H, _BK), lambda o, k, i: (0, k)),
            pl.BlockSpec((1, _H), lambda o, k, i: (0, 0)),
            pl.BlockSpec((_H, _H), lambda o, k, i: (0, 0)),
            pl.BlockSpec((1, _H), lambda o, k, i: (0, 0)),
            pl.BlockSpec((c1, _H), lambda o, k, i: (0, 0)),
            pl.BlockSpec((1, c1), lambda o, k, i: (0, 0)),
            pl.BlockSpec((c4, _H), lambda o, k, i: (0, 0)),
            pl.BlockSpec((1, c4), lambda o, k, i: (0, 0)),
        ],
        out_specs=[
            pl.BlockSpec((_BN, c1), lambda o, k, i: (o * _NI + i, 0)),
            pl.BlockSpec((_BN, c4), lambda o, k, i: (o * _NI + i, 0)),
        ],
        out_shape=[
            jax.ShapeDtypeStruct((_N, c1), jnp.float32),
            jax.ShapeDtypeStruct((_N, c4), jnp.float32),
        ],
        scratch_shapes=[pltpu.VMEM((_NI, _BN, _H), jnp.float32)],
        compiler_params=pltpu.CompilerParams(
            dimension_semantics=("arbitrary", "arbitrary", "arbitrary")),
    )(feature_vectors, W1, b1.reshape(1, -1), W2.astype(jnp.bfloat16),
      b2.reshape(1, -1), Wc.astype(jnp.bfloat16), bc.reshape(1, -1),
      Wr.astype(jnp.bfloat16), br.reshape(1, -1))
    return (cls_out, box_out)
